# Initial kernel scaffold; baseline (speedup 1.0000x reference)
import jax
import jax.numpy as jnp
from jax.experimental import pallas as pl


def kernel(x, emb_weight):
    raise NotImplementedError("write your pallas kernel here")


# speedup vs baseline: 4.0919x; 4.0919x over previous
import functools

import jax
import jax.numpy as jnp
from jax import lax
from jax.experimental import pallas as pl
from jax.experimental.pallas import tpu as pltpu
from jax.experimental.pallas import tpu_sc as plsc

_INFO = plsc.get_sparse_core_info()
_NC = _INFO.num_cores
_NS = _INFO.num_subcores
_NW = _NC * _NS

_CH = 128


def _gather_rows(table, idx_flat, n, d):
    bpw = n // _NW
    g = bpw // _CH

    mesh = plsc.VectorSubcoreMesh(core_axis_name="c", subcore_axis_name="s")

    @functools.partial(
        pl.kernel,
        mesh=mesh,
        out_type=jax.ShapeDtypeStruct((n, d), jnp.float32),
        scratch_types=[
            pltpu.VMEM((bpw,), jnp.int32),
            pltpu.VMEM((_CH, d), jnp.float32),
            pltpu.SemaphoreType.DMA,
        ],
        compiler_params=pltpu.CompilerParams(use_tc_tiling_on_sc=False),
    )
    def k(table_hbm, idx_hbm, out_hbm, idx_v, rows_v, sem):
        wid = lax.axis_index("s") * _NC + lax.axis_index("c")
        base = wid * bpw
        pltpu.sync_copy(idx_hbm.at[pl.ds(base, bpw)], idx_v)

        def step(c, carry):
            off = c * _CH
            pltpu.async_copy(
                table_hbm.at[idx_v.at[pl.ds(off, _CH)]], rows_v, sem
            ).wait()
            pltpu.sync_copy(rows_v, out_hbm.at[pl.ds(base + off, _CH)])
            return carry

        lax.fori_loop(0, g, step, 0)

    return k(table, idx_flat)


def kernel(x, emb_weight):
    b, s = x.shape
    v, d = emb_weight.shape
    n = b * s
    idx_flat = x.reshape(n).astype(jnp.int32)
    out = _gather_rows(emb_weight, idx_flat, n, d)
    return out.reshape(b, s, d)


# speedup vs baseline: 4.6182x; 1.1286x over previous
import functools

import jax
import jax.numpy as jnp
from jax import lax
from jax.experimental import pallas as pl
from jax.experimental.pallas import tpu as pltpu
from jax.experimental.pallas import tpu_sc as plsc

_INFO = plsc.get_sparse_core_info()
_NC = _INFO.num_cores
_NS = _INFO.num_subcores
_NW = _NC * _NS

_CH = 128
_NBUF = 5


def _gather_rows(table, idx_flat, n, d):
    bpw = n // _NW
    g = bpw // _CH
    assert g % _NBUF == 0

    mesh = plsc.VectorSubcoreMesh(core_axis_name="c", subcore_axis_name="s")

    @functools.partial(
        pl.kernel,
        mesh=mesh,
        out_type=jax.ShapeDtypeStruct((n, d), jnp.float32),
        scratch_types=(
            [pltpu.VMEM((bpw,), jnp.int32),
             pltpu.VMEM((_NBUF, _CH, d), jnp.float32)]
            + [pltpu.SemaphoreType.DMA] * (2 * _NBUF)
        ),
        compiler_params=pltpu.CompilerParams(use_tc_tiling_on_sc=False),
    )
    def k(table_hbm, idx_hbm, out_hbm, idx_v, rows_v, *sems):
        gsem = sems[:_NBUF]
        ssem = sems[_NBUF:]
        wid = lax.axis_index("s") * _NC + lax.axis_index("c")
        base = wid * bpw
        pltpu.sync_copy(idx_hbm.at[pl.ds(base, bpw)], idx_v)

        def start_gather(b, c):
            pltpu.async_copy(
                table_hbm.at[idx_v.at[pl.ds(c * _CH, _CH)]], rows_v.at[b], gsem[b]
            )

        def wait_gather(b):
            pltpu.make_async_copy(
                table_hbm.at[pl.ds(0, _CH)], rows_v.at[b], gsem[b]
            ).wait()

        def start_store(b, c):
            pltpu.async_copy(
                rows_v.at[b], out_hbm.at[pl.ds(base + c * _CH, _CH)], ssem[b]
            )

        def wait_store(b):
            pltpu.make_async_copy(
                rows_v.at[b], out_hbm.at[pl.ds(0, _CH)], ssem[b]
            ).wait()

        for b in range(_NBUF):
            start_gather(b, b)

        def outer(i, carry):
            c0 = i * _NBUF
            for b in range(_NBUF):
                c = c0 + b
                wait_gather(b)
                start_store(b, c)
            for b in range(_NBUF):
                wait_store(b)
                nxt = jnp.minimum(c0 + b + _NBUF, g - 1)
                start_gather(b, nxt)
            return carry

        lax.fori_loop(0, g // _NBUF, outer, 0)

        for b in range(_NBUF):
            wait_gather(b)

    return k(table, idx_flat)


def kernel(x, emb_weight):
    b, s = x.shape
    v, d = emb_weight.shape
    n = b * s
    idx_flat = x.reshape(n).astype(jnp.int32)
    out = _gather_rows(emb_weight, idx_flat, n, d)
    return out.reshape(b, s, d)
